# baseline (speedup 1.0000x reference)
import functools

import numpy as np
import jax
import jax.numpy as jnp
from jax import lax
from jax.experimental import pallas as pl
from jax.experimental.pallas import tpu as pltpu
from jax.experimental.pallas import tpu_sc as plsc

N = 10000
K = 32
C = 128
L = 16
NQ = C // 32
NW = 32
RPW = 320
NP = NW * RPW
CH = 4
NCH = RPW // CH
NBUF = 2
SEG = NP // 16

_PERM = np.empty((C,), dtype=np.int32)
for _q in range(NQ):
    for _i in range(16):
        _PERM[32 * _q + 2 * _i] = 32 * _q + _i
        _PERM[32 * _q + 2 * _i + 1] = 32 * _q + 16 + _i


def _mm_body(x_ref, w_ref, w2p_ref, b_ref, a_ref, t_ref):
    xb = x_ref[...]
    w = w_ref[...]
    wd = w[:C, :] - w[C:, :]
    a_ref[...] = jnp.dot(xb, wd, preferred_element_type=jnp.float32) + b_ref[...]
    t = jnp.dot(xb, w2p_ref[...], preferred_element_type=jnp.float32)
    t_ref[...] = t.astype(jnp.bfloat16)


def _tc_matmul(x_pad, W, W2p, b2d):
    BLK = 1024
    return pl.pallas_call(
        _mm_body,
        grid=(NP // BLK,),
        in_specs=[
            pl.BlockSpec((BLK, C), lambda i: (i, 0)),
            pl.BlockSpec((2 * C, C), lambda i: (0, 0)),
            pl.BlockSpec((C, C), lambda i: (0, 0)),
            pl.BlockSpec((1, C), lambda i: (0, 0)),
        ],
        out_specs=[
            pl.BlockSpec((BLK, C), lambda i: (i, 0)),
            pl.BlockSpec((BLK, C), lambda i: (i, 0)),
        ],
        out_shape=[
            jax.ShapeDtypeStruct((NP, C), jnp.float32),
            jax.ShapeDtypeStruct((NP, C), jnp.bfloat16),
        ],
    )(x_pad, W, W2p, b2d)


def _elu(v):
    return jnp.where(v > 0.0, v, jnp.exp(v) - 1.0)


def _sc_body(idx_hbm, a_hbm, tab_hbm, out_hbm, idx_v, rows_v, a_b, out_b,
             tab_sh, sem_g0, sem_g1, sem_a0, sem_a1, sem_s0, sem_s1):
    cid = lax.axis_index("c")
    sid = lax.axis_index("s")
    wid = sid * 2 + cid
    rbase = wid * RPW
    pltpu.sync_copy(tab_hbm.at[pl.ds(sid * SEG, SEG)],
                    tab_sh.at[pl.ds(sid * SEG, SEG)])
    pltpu.sync_copy(idx_hbm.at[pl.ds(rbase * K, RPW * K)], idx_v)
    plsc.subcore_barrier()
    sems_g = (sem_g0, sem_g1)
    sems_a = (sem_a0, sem_a1)
    sems_s = (sem_s0, sem_s1)

    def gstart(ck, buf):
        pltpu.make_async_copy(
            tab_sh.at[idx_v.at[pl.ds(ck * (CH * K), CH * K)]],
            rows_v.at[buf],
            sems_g[buf],
        ).start()

    def gwait(buf):
        pltpu.make_async_copy(
            tab_sh.at[idx_v.at[pl.ds(0, CH * K)]],
            rows_v.at[buf],
            sems_g[buf],
        ).wait()

    def astart(ck, buf):
        pltpu.make_async_copy(
            a_hbm.at[pl.ds(rbase + ck * CH, CH)],
            a_b.at[buf],
            sems_a[buf],
        ).start()

    def await_(buf):
        pltpu.make_async_copy(
            a_hbm.at[pl.ds(rbase, CH)],
            a_b.at[buf],
            sems_a[buf],
        ).wait()

    def sstart(ck, buf):
        pltpu.make_async_copy(
            out_b.at[buf],
            out_hbm.at[pl.ds(rbase + ck * CH, CH)],
            sems_s[buf],
        ).start()

    def swait(buf):
        pltpu.make_async_copy(
            out_b.at[buf],
            out_hbm.at[pl.ds(rbase, CH)],
            sems_s[buf],
        ).wait()

    for buf in range(NBUF):
        gstart(buf, buf)
        astart(buf, buf)

    def step(ckg, carry):
        for buf in range(NBUF):
            ck = ckg * NBUF + buf
            gwait(buf)
            await_(buf)

            @pl.when(ck >= NBUF)
            def _(_buf=buf):
                swait(_buf)

            for nloc in range(CH):
                base = nloc * K
                acc_lo = [None] * NQ
                acc_hi = [None] * NQ
                for j in range(K):
                    for q in range(NQ):
                        pw = rows_v[buf, base + j, pl.ds(q * L, L)]
                        lo = lax.bitcast_convert_type(lax.shift_left(pw, 16), jnp.float32)
                        hi = lax.bitcast_convert_type(pw, jnp.float32)
                        if j == 0:
                            acc_lo[q], acc_hi[q] = lo, hi
                        else:
                            acc_lo[q] = jnp.maximum(acc_lo[q], lo)
                            acc_hi[q] = jnp.maximum(acc_hi[q], hi)
                for q in range(NQ):
                    hi_clean = lax.bitcast_convert_type(
                        lax.bitwise_and(
                            lax.bitcast_convert_type(acc_hi[q], jnp.int32),
                            jnp.int32(-65536)), jnp.float32)
                    vlo = acc_lo[q] + a_b[buf, nloc, pl.ds(q * 32, L)]
                    vhi = hi_clean + a_b[buf, nloc, pl.ds(q * 32 + L, L)]
                    out_b[buf, nloc, pl.ds(q * 32, L)] = _elu(vlo)
                    out_b[buf, nloc, pl.ds(q * 32 + L, L)] = _elu(vhi)

            sstart(ck, buf)

            @pl.when(ck + NBUF < NCH)
            def _(_ck=ck, _buf=buf):
                gstart(_ck + NBUF, _buf)
                astart(_ck + NBUF, _buf)

        return carry

    lax.fori_loop(0, NCH // NBUF, step, 0)
    for buf in range(NBUF):
        swait(buf)


_sc_gather_max = pl.kernel(
    _sc_body,
    out_type=jax.ShapeDtypeStruct((NP, C), jnp.float32),
    mesh=plsc.VectorSubcoreMesh(core_axis_name="c", subcore_axis_name="s"),
    compiler_params=pltpu.CompilerParams(use_tc_tiling_on_sc=False),
    scratch_types=[
        pltpu.VMEM((RPW * K,), jnp.int32),
        pltpu.VMEM((NBUF, CH * K, C // 2), jnp.int32),
        pltpu.VMEM((NBUF, CH, C), jnp.float32),
        pltpu.VMEM((NBUF, CH, C), jnp.float32),
        pltpu.VMEM_SHARED((NP, C // 2), jnp.int32),
    ] + [pltpu.SemaphoreType.DMA] * 6,
)


def kernel(x, edge_index, W, b):
    x2 = x[0]
    x_pad = jnp.concatenate([x2, jnp.zeros((NP - N, C), x.dtype)], axis=0)
    W2p = W[C:, _PERM]
    a_full, tab_bf = _tc_matmul(x_pad, W, W2p, b.reshape(1, C))
    tab = lax.bitcast_convert_type(tab_bf.reshape(NP, C // 2, 2), jnp.int32)
    eflat = edge_index[0].reshape(N * K)
    e_pad = jnp.concatenate(
        [eflat, jnp.zeros(((NP - N) * K,), jnp.int32)], axis=0)
    out = _sc_gather_max(e_pad, a_full, tab)
    return out[:N].reshape(1, N, C)


# speedup vs baseline: 4.0018x; 4.0018x over previous
import functools

import numpy as np
import jax
import jax.numpy as jnp
from jax import lax
from jax.experimental import pallas as pl
from jax.experimental.pallas import tpu as pltpu
from jax.experimental.pallas import tpu_sc as plsc

N = 10000
K = 32
C = 128
L = 16
NQ = C // 32
NW = 32
RPW = 320
NP = NW * RPW
CH = 4
NCH = RPW // CH
NBUF = 2
SEG = NP // 16

_PERM = np.empty((C,), dtype=np.int32)
for _q in range(NQ):
    for _i in range(16):
        _PERM[32 * _q + 2 * _i] = 32 * _q + _i
        _PERM[32 * _q + 2 * _i + 1] = 32 * _q + 16 + _i
_PMAT = np.zeros((C, C), dtype=np.float32)
for _j in range(C):
    _PMAT[_PERM[_j], _j] = 1.0


def _mm_body(x_ref, w_ref, w2p_ref, b_ref, a_ref, t_ref):
    xb = x_ref[...]
    w = w_ref[...]
    wd = w[:C, :] - w[C:, :]
    a_ref[...] = jnp.dot(xb, wd, preferred_element_type=jnp.float32) + b_ref[...]
    t = jnp.dot(xb, w2p_ref[...], preferred_element_type=jnp.float32)
    t_ref[...] = t.astype(jnp.bfloat16)


def _tc_matmul(x_pad, W, W2p, b2d):
    BLK = 1024
    return pl.pallas_call(
        _mm_body,
        grid=(NP // BLK,),
        in_specs=[
            pl.BlockSpec((BLK, C), lambda i: (i, 0)),
            pl.BlockSpec((2 * C, C), lambda i: (0, 0)),
            pl.BlockSpec((C, C), lambda i: (0, 0)),
            pl.BlockSpec((1, C), lambda i: (0, 0)),
        ],
        out_specs=[
            pl.BlockSpec((BLK, C), lambda i: (i, 0)),
            pl.BlockSpec((BLK, C), lambda i: (i, 0)),
        ],
        out_shape=[
            jax.ShapeDtypeStruct((NP, C), jnp.float32),
            jax.ShapeDtypeStruct((NP, C), jnp.bfloat16),
        ],
    )(x_pad, W, W2p, b2d)


def _elu(v):
    return jnp.where(v > 0.0, v, jnp.exp(v) - 1.0)


def _sc_body(idx_hbm, a_hbm, tab_hbm, out_hbm, idx_v, rows_v, a_b, out_b,
             tab_sh, sem_g0, sem_g1, sem_a0, sem_a1, sem_s0, sem_s1):
    cid = lax.axis_index("c")
    sid = lax.axis_index("s")
    wid = sid * 2 + cid
    rbase = wid * RPW
    pltpu.sync_copy(tab_hbm.at[pl.ds(sid * SEG, SEG)],
                    tab_sh.at[pl.ds(sid * SEG, SEG)])
    pltpu.sync_copy(idx_hbm.at[pl.ds(rbase * K, RPW * K)], idx_v)
    plsc.subcore_barrier()
    sems_g = (sem_g0, sem_g1)
    sems_a = (sem_a0, sem_a1)
    sems_s = (sem_s0, sem_s1)

    def gstart(ck, buf):
        pltpu.make_async_copy(
            tab_sh.at[idx_v.at[pl.ds(ck * (CH * K), CH * K)]],
            rows_v.at[buf],
            sems_g[buf],
        ).start()

    def gwait(buf):
        pltpu.make_async_copy(
            tab_sh.at[idx_v.at[pl.ds(0, CH * K)]],
            rows_v.at[buf],
            sems_g[buf],
        ).wait()

    def astart(ck, buf):
        pltpu.make_async_copy(
            a_hbm.at[pl.ds(rbase + ck * CH, CH)],
            a_b.at[buf],
            sems_a[buf],
        ).start()

    def await_(buf):
        pltpu.make_async_copy(
            a_hbm.at[pl.ds(rbase, CH)],
            a_b.at[buf],
            sems_a[buf],
        ).wait()

    def sstart(ck, buf):
        pltpu.make_async_copy(
            out_b.at[buf],
            out_hbm.at[pl.ds(rbase + ck * CH, CH)],
            sems_s[buf],
        ).start()

    def swait(buf):
        pltpu.make_async_copy(
            out_b.at[buf],
            out_hbm.at[pl.ds(rbase, CH)],
            sems_s[buf],
        ).wait()

    for buf in range(NBUF):
        gstart(buf, buf)
        astart(buf, buf)

    def step(ckg, carry):
        for buf in range(NBUF):
            ck = ckg * NBUF + buf
            gwait(buf)
            await_(buf)

            @pl.when(ck >= NBUF)
            def _(_buf=buf):
                swait(_buf)

            for nloc in range(CH):
                base = nloc * K
                acc_lo = [None] * NQ
                acc_hi = [None] * NQ
                for j in range(K):
                    for q in range(NQ):
                        pw = rows_v[buf, base + j, pl.ds(q * L, L)]
                        lo = lax.bitcast_convert_type(lax.shift_left(pw, 16), jnp.float32)
                        hi = lax.bitcast_convert_type(pw, jnp.float32)
                        if j == 0:
                            acc_lo[q], acc_hi[q] = lo, hi
                        else:
                            acc_lo[q] = jnp.maximum(acc_lo[q], lo)
                            acc_hi[q] = jnp.maximum(acc_hi[q], hi)
                for q in range(NQ):
                    hi_clean = lax.bitcast_convert_type(
                        lax.bitwise_and(
                            lax.bitcast_convert_type(acc_hi[q], jnp.int32),
                            jnp.int32(-65536)), jnp.float32)
                    vlo = acc_lo[q] + a_b[buf, nloc, pl.ds(q * 32, L)]
                    vhi = hi_clean + a_b[buf, nloc, pl.ds(q * 32 + L, L)]
                    out_b[buf, nloc, pl.ds(q * 32, L)] = _elu(vlo)
                    out_b[buf, nloc, pl.ds(q * 32 + L, L)] = _elu(vhi)

            sstart(ck, buf)

            @pl.when(ck + NBUF < NCH)
            def _(_ck=ck, _buf=buf):
                gstart(_ck + NBUF, _buf)
                astart(_ck + NBUF, _buf)

        return carry

    lax.fori_loop(0, NCH // NBUF, step, 0)
    for buf in range(NBUF):
        swait(buf)


_sc_gather_max = pl.kernel(
    _sc_body,
    out_type=jax.ShapeDtypeStruct((NP, C), jnp.float32),
    mesh=plsc.VectorSubcoreMesh(core_axis_name="c", subcore_axis_name="s"),
    compiler_params=pltpu.CompilerParams(use_tc_tiling_on_sc=False),
    scratch_types=[
        pltpu.VMEM((RPW * K,), jnp.int32),
        pltpu.VMEM((NBUF, CH * K, C // 2), jnp.int32),
        pltpu.VMEM((NBUF, CH, C), jnp.float32),
        pltpu.VMEM((NBUF, CH, C), jnp.float32),
        pltpu.VMEM_SHARED((NP, C // 2), jnp.int32),
    ] + [pltpu.SemaphoreType.DMA] * 6,
)


def kernel(x, edge_index, W, b):
    x2 = x[0]
    x_pad = jnp.concatenate([x2, jnp.zeros((NP - N, C), x.dtype)], axis=0)
    W2p = lax.dot(W[C:], jnp.asarray(_PMAT),
                  precision=lax.Precision.HIGHEST)
    a_full, tab_bf = _tc_matmul(x_pad, W, W2p, b.reshape(1, C))
    tab = lax.bitcast_convert_type(tab_bf.reshape(NP, C // 2, 2), jnp.int32)
    eflat = edge_index[0].reshape(N * K)
    e_pad = jnp.concatenate(
        [eflat, jnp.zeros(((NP - N) * K,), jnp.int32)], axis=0)
    out = _sc_gather_max(e_pad, a_full, tab)
    return out[:N].reshape(1, N, C)


# speedup vs baseline: 5.2923x; 1.3225x over previous
import functools

import numpy as np
import jax
import jax.numpy as jnp
from jax import lax
from jax.experimental import pallas as pl
from jax.experimental.pallas import tpu as pltpu
from jax.experimental.pallas import tpu_sc as plsc

N = 10000
K = 32
C = 128
L = 16
NQ = C // 32
NW = 32
RPW = 320
NP = NW * RPW
CH = 4
NCH = RPW // CH
NBUF = 2
SEG = NP // 16

_PERM = np.empty((C,), dtype=np.int32)
for _q in range(NQ):
    for _i in range(16):
        _PERM[16 * _q + _i] = 32 * _q + _i
        _PERM[C // 2 + 16 * _q + _i] = 32 * _q + 16 + _i
_PMAT = np.zeros((C, C), dtype=np.float32)
for _j in range(C):
    _PMAT[_PERM[_j], _j] = 1.0


def _mm_body(x_ref, w_ref, w2p_ref, b_ref, a_ref, t_ref):
    xb = x_ref[...]
    w = w_ref[...]
    wd = w[:C, :] - w[C:, :]
    a_ref[...] = jnp.dot(xb, wd, preferred_element_type=jnp.float32) + b_ref[...]
    t = jnp.dot(xb, w2p_ref[...], preferred_element_type=jnp.float32)
    rlo = lax.bitcast_convert_type(
        t[:, :C // 2].astype(jnp.bfloat16).astype(jnp.float32), jnp.int32)
    rhi = lax.bitcast_convert_type(
        t[:, C // 2:].astype(jnp.bfloat16).astype(jnp.float32), jnp.int32)
    t_ref[...] = rhi | lax.shift_right_logical(rlo, 16)


def _tc_matmul(x_pad, W, W2p, b2d):
    BLK = 1024
    return pl.pallas_call(
        _mm_body,
        grid=(NP // BLK,),
        in_specs=[
            pl.BlockSpec((BLK, C), lambda i: (i, 0)),
            pl.BlockSpec((2 * C, C), lambda i: (0, 0)),
            pl.BlockSpec((C, C), lambda i: (0, 0)),
            pl.BlockSpec((1, C), lambda i: (0, 0)),
        ],
        out_specs=[
            pl.BlockSpec((BLK, C), lambda i: (i, 0)),
            pl.BlockSpec((BLK, C // 2), lambda i: (i, 0)),
        ],
        out_shape=[
            jax.ShapeDtypeStruct((NP, C), jnp.float32),
            jax.ShapeDtypeStruct((NP, C // 2), jnp.int32),
        ],
    )(x_pad, W, W2p, b2d)


def _elu(v):
    return jnp.where(v > 0.0, v, jnp.exp(v) - 1.0)


def _sc_body(idx_hbm, a_hbm, tab_hbm, out_hbm, idx_v, rows_v, a_b, out_b,
             tab_sh, sem_g0, sem_g1, sem_a0, sem_a1, sem_s0, sem_s1):
    cid = lax.axis_index("c")
    sid = lax.axis_index("s")
    wid = sid * 2 + cid
    rbase = wid * RPW
    pltpu.sync_copy(tab_hbm.at[pl.ds(sid * SEG, SEG)],
                    tab_sh.at[pl.ds(sid * SEG, SEG)])
    pltpu.sync_copy(idx_hbm.at[pl.ds(rbase * K, RPW * K)], idx_v)
    plsc.subcore_barrier()
    sems_g = (sem_g0, sem_g1)
    sems_a = (sem_a0, sem_a1)
    sems_s = (sem_s0, sem_s1)

    def gstart(ck, buf):
        pltpu.make_async_copy(
            tab_sh.at[idx_v.at[pl.ds(ck * (CH * K), CH * K)]],
            rows_v.at[buf],
            sems_g[buf],
        ).start()

    def gwait(buf):
        pltpu.make_async_copy(
            tab_sh.at[idx_v.at[pl.ds(0, CH * K)]],
            rows_v.at[buf],
            sems_g[buf],
        ).wait()

    def astart(ck, buf):
        pltpu.make_async_copy(
            a_hbm.at[pl.ds(rbase + ck * CH, CH)],
            a_b.at[buf],
            sems_a[buf],
        ).start()

    def await_(buf):
        pltpu.make_async_copy(
            a_hbm.at[pl.ds(rbase, CH)],
            a_b.at[buf],
            sems_a[buf],
        ).wait()

    def sstart(ck, buf):
        pltpu.make_async_copy(
            out_b.at[buf],
            out_hbm.at[pl.ds(rbase + ck * CH, CH)],
            sems_s[buf],
        ).start()

    def swait(buf):
        pltpu.make_async_copy(
            out_b.at[buf],
            out_hbm.at[pl.ds(rbase, CH)],
            sems_s[buf],
        ).wait()

    for buf in range(NBUF):
        gstart(buf, buf)
        astart(buf, buf)

    def step(ckg, carry):
        for buf in range(NBUF):
            ck = ckg * NBUF + buf
            gwait(buf)
            await_(buf)

            @pl.when(ck >= NBUF)
            def _(_buf=buf):
                swait(_buf)

            for nloc in range(CH):
                base = nloc * K
                acc_lo = [None] * NQ
                acc_hi = [None] * NQ
                for j in range(K):
                    for q in range(NQ):
                        pw = rows_v[buf, base + j, pl.ds(q * L, L)]
                        lo = lax.bitcast_convert_type(lax.shift_left(pw, 16), jnp.float32)
                        hi = lax.bitcast_convert_type(pw, jnp.float32)
                        if j == 0:
                            acc_lo[q], acc_hi[q] = lo, hi
                        else:
                            acc_lo[q] = jnp.maximum(acc_lo[q], lo)
                            acc_hi[q] = jnp.maximum(acc_hi[q], hi)
                for q in range(NQ):
                    hi_clean = lax.bitcast_convert_type(
                        lax.bitwise_and(
                            lax.bitcast_convert_type(acc_hi[q], jnp.int32),
                            jnp.int32(-65536)), jnp.float32)
                    vlo = acc_lo[q] + a_b[buf, nloc, pl.ds(q * 32, L)]
                    vhi = hi_clean + a_b[buf, nloc, pl.ds(q * 32 + L, L)]
                    out_b[buf, nloc, pl.ds(q * 32, L)] = _elu(vlo)
                    out_b[buf, nloc, pl.ds(q * 32 + L, L)] = _elu(vhi)

            sstart(ck, buf)

            @pl.when(ck + NBUF < NCH)
            def _(_ck=ck, _buf=buf):
                gstart(_ck + NBUF, _buf)
                astart(_ck + NBUF, _buf)

        return carry

    lax.fori_loop(0, NCH // NBUF, step, 0)
    for buf in range(NBUF):
        swait(buf)


_sc_gather_max = pl.kernel(
    _sc_body,
    out_type=jax.ShapeDtypeStruct((NP, C), jnp.float32),
    mesh=plsc.VectorSubcoreMesh(core_axis_name="c", subcore_axis_name="s"),
    scratch_types=[
        pltpu.VMEM((RPW * K,), jnp.int32),
        pltpu.VMEM((NBUF, CH * K, C // 2), jnp.int32),
        pltpu.VMEM((NBUF, CH, C), jnp.float32),
        pltpu.VMEM((NBUF, CH, C), jnp.float32),
        pltpu.VMEM_SHARED((NP, C // 2), jnp.int32),
    ] + [pltpu.SemaphoreType.DMA] * 6,
)


def kernel(x, edge_index, W, b):
    x2 = x[0]
    x_pad = jnp.concatenate([x2, jnp.zeros((NP - N, C), x.dtype)], axis=0)
    W2p = lax.dot(W[C:], jnp.asarray(_PMAT),
                  precision=lax.Precision.HIGHEST)
    a_full, tab = _tc_matmul(x_pad, W, W2p, b.reshape(1, C))
    eflat = edge_index[0].reshape(N * K)
    e_pad = jnp.concatenate(
        [eflat, jnp.zeros(((NP - N) * K,), jnp.int32)], axis=0)
    out = _sc_gather_max(e_pad, a_full, tab)
    return out[:N].reshape(1, N, C)


# speedup vs baseline: 5.3041x; 1.0022x over previous
import functools

import numpy as np
import jax
import jax.numpy as jnp
from jax import lax
from jax.experimental import pallas as pl
from jax.experimental.pallas import tpu as pltpu
from jax.experimental.pallas import tpu_sc as plsc

N = 10000
K = 32
C = 128
L = 16
NQ = C // 32
NW = 32
RPW = 320
NP = NW * RPW
CH = 4
NCH = RPW // CH
NBUF = 2
SEG = NP // 16
PAD = NP - N

_PERM = np.empty((C,), dtype=np.int32)
for _q in range(NQ):
    for _i in range(16):
        _PERM[16 * _q + _i] = 32 * _q + _i
        _PERM[C // 2 + 16 * _q + _i] = 32 * _q + 16 + _i
_PMAT = np.zeros((C, C), dtype=np.float32)
for _j in range(C):
    _PMAT[_PERM[_j], _j] = 1.0


def _mm_body(x_ref, w_ref, w2p_ref, b_ref, a_ref, t_ref):
    xb = x_ref[...]
    w = w_ref[...]
    wd = w[:C, :] - w[C:, :]
    a_ref[...] = jnp.dot(xb, wd, preferred_element_type=jnp.float32) + b_ref[...]
    t = jnp.dot(xb, w2p_ref[...], preferred_element_type=jnp.float32)
    rlo = lax.bitcast_convert_type(
        t[:, :C // 2].astype(jnp.bfloat16).astype(jnp.float32), jnp.int32)
    rhi = lax.bitcast_convert_type(
        t[:, C // 2:].astype(jnp.bfloat16).astype(jnp.float32), jnp.int32)
    t_ref[...] = rhi | lax.shift_right_logical(rlo, 16)


def _tc_matmul(x_pad, W, W2p, b2d):
    BLK = 1024
    return pl.pallas_call(
        _mm_body,
        grid=(NP // BLK,),
        in_specs=[
            pl.BlockSpec((BLK, C), lambda i: (i, 0)),
            pl.BlockSpec((2 * C, C), lambda i: (0, 0)),
            pl.BlockSpec((C, C), lambda i: (0, 0)),
            pl.BlockSpec((1, C), lambda i: (0, 0)),
        ],
        out_specs=[
            pl.BlockSpec((BLK, C), lambda i: (i, 0)),
            pl.BlockSpec((BLK, C // 2), lambda i: (i, 0)),
        ],
        out_shape=[
            jax.ShapeDtypeStruct((NP, C), jnp.float32),
            jax.ShapeDtypeStruct((NP, C // 2), jnp.int32),
        ],
    )(x_pad, W, W2p, b2d)


def _elu(v):
    return jnp.where(v > 0.0, v, jnp.exp(v) - 1.0)


def _sc_body(idx_hbm, a_hbm, tab_hbm, out_hbm, idx_v, rows_v, a_b, out_b,
             tab_sh, sem_g0, sem_g1, sem_a0, sem_a1, sem_s0, sem_s1):
    cid = lax.axis_index("c")
    sid = lax.axis_index("s")
    wid = sid * 2 + cid
    rbase = wid * RPW
    pltpu.sync_copy(tab_hbm.at[pl.ds(sid * SEG, SEG)],
                    tab_sh.at[pl.ds(sid * SEG, SEG)])
    pltpu.sync_copy(idx_hbm.at[pl.ds(rbase * K, RPW * K)], idx_v)
    plsc.subcore_barrier()
    sems_g = (sem_g0, sem_g1)
    sems_a = (sem_a0, sem_a1)
    sems_s = (sem_s0, sem_s1)

    def gstart(ck, buf):
        pltpu.make_async_copy(
            tab_sh.at[idx_v.at[pl.ds(ck * (CH * K), CH * K)]],
            rows_v.at[buf],
            sems_g[buf],
        ).start()

    def gwait(buf):
        pltpu.make_async_copy(
            tab_sh.at[idx_v.at[pl.ds(0, CH * K)]],
            rows_v.at[buf],
            sems_g[buf],
        ).wait()

    def astart(ck, buf):
        pltpu.make_async_copy(
            a_hbm.at[pl.ds(rbase + ck * CH, CH)],
            a_b.at[buf],
            sems_a[buf],
        ).start()

    def await_(buf):
        pltpu.make_async_copy(
            a_hbm.at[pl.ds(rbase, CH)],
            a_b.at[buf],
            sems_a[buf],
        ).wait()

    def sstart(ck, buf):
        row = rbase + ck * CH
        row = row - jnp.where(row + CH > N, PAD, 0)
        pltpu.make_async_copy(
            out_b.at[buf],
            out_hbm.at[pl.ds(row, CH)],
            sems_s[buf],
        ).start()

    def swait(buf):
        pltpu.make_async_copy(
            out_b.at[buf],
            out_hbm.at[pl.ds(rbase, CH)],
            sems_s[buf],
        ).wait()

    for buf in range(NBUF):
        gstart(buf, buf)
        astart(buf, buf)

    def step(ckg, carry):
        for buf in range(NBUF):
            ck = ckg * NBUF + buf
            gwait(buf)
            await_(buf)

            @pl.when(ck >= NBUF)
            def _(_buf=buf):
                swait(_buf)

            for nloc in range(CH):
                base = nloc * K
                acc_lo = [None] * NQ
                acc_hi = [None] * NQ
                for j in range(K):
                    for q in range(NQ):
                        pw = rows_v[buf, base + j, pl.ds(q * L, L)]
                        lo = lax.bitcast_convert_type(lax.shift_left(pw, 16), jnp.float32)
                        hi = lax.bitcast_convert_type(pw, jnp.float32)
                        if j == 0:
                            acc_lo[q], acc_hi[q] = lo, hi
                        else:
                            acc_lo[q] = jnp.maximum(acc_lo[q], lo)
                            acc_hi[q] = jnp.maximum(acc_hi[q], hi)
                for q in range(NQ):
                    hi_clean = lax.bitcast_convert_type(
                        lax.bitwise_and(
                            lax.bitcast_convert_type(acc_hi[q], jnp.int32),
                            jnp.int32(-65536)), jnp.float32)
                    vlo = acc_lo[q] + a_b[buf, nloc, pl.ds(q * 32, L)]
                    vhi = hi_clean + a_b[buf, nloc, pl.ds(q * 32 + L, L)]
                    out_b[buf, nloc, pl.ds(q * 32, L)] = _elu(vlo)
                    out_b[buf, nloc, pl.ds(q * 32 + L, L)] = _elu(vhi)

            sstart(ck, buf)

            @pl.when(ck + NBUF < NCH)
            def _(_ck=ck, _buf=buf):
                gstart(_ck + NBUF, _buf)
                astart(_ck + NBUF, _buf)

        return carry

    lax.fori_loop(0, NCH // NBUF, step, 0)
    for buf in range(NBUF):
        swait(buf)


_sc_gather_max = pl.kernel(
    _sc_body,
    out_type=jax.ShapeDtypeStruct((N, C), jnp.float32),
    mesh=plsc.VectorSubcoreMesh(core_axis_name="c", subcore_axis_name="s"),
    scratch_types=[
        pltpu.VMEM((RPW * K,), jnp.int32),
        pltpu.VMEM((NBUF, CH * K, C // 2), jnp.int32),
        pltpu.VMEM((NBUF, CH, C), jnp.float32),
        pltpu.VMEM((NBUF, CH, C), jnp.float32),
        pltpu.VMEM_SHARED((NP, C // 2), jnp.int32),
    ] + [pltpu.SemaphoreType.DMA] * 6,
)


def kernel(x, edge_index, W, b):
    x2 = x[0]
    x_pad = jnp.concatenate([x2, x2[N - PAD:]], axis=0)
    W2p = lax.dot(W[C:], jnp.asarray(_PMAT),
                  precision=lax.Precision.HIGHEST)
    a_full, tab = _tc_matmul(x_pad, W, W2p, b.reshape(1, C))
    eflat = edge_index[0].reshape(N * K)
    e_pad = jnp.concatenate([eflat, eflat[(N - PAD) * K:]], axis=0)
    out = _sc_gather_max(e_pad, a_full, tab)
    return out.reshape(1, N, C)
